# Initial kernel scaffold; baseline (speedup 1.0000x reference)
import jax
import jax.numpy as jnp
from jax.experimental import pallas as pl


def kernel(adjacency):
    raise NotImplementedError("write your pallas kernel here")


# speedup vs baseline: 2.4646x; 2.4646x over previous
import jax
import jax.numpy as jnp
from jax.experimental import pallas as pl
from jax.experimental.pallas import tpu as pltpu

_N = 10000
_LAM = 1.0
_R = 200


def _body(a_ref, o_ref):
    i = pl.program_id(0)
    a = a_ref[...]
    rs = jnp.sum(a, axis=1, keepdims=True)
    den = rs + 1.0
    inv = jnp.where(den == 0.0, 0.0, 1.0 / den)
    rows = jax.lax.broadcasted_iota(jnp.int32, a.shape, 0) + i * _R
    cols = jax.lax.broadcasted_iota(jnp.int32, a.shape, 1)
    m = (rows == cols).astype(a.dtype)
    o_ref[...] = (inv * (a + m)) * (1.0 + _LAM * m)


def kernel(adjacency):
    adjacency = adjacency.astype(jnp.float32)
    n = adjacency.shape[0]
    grid = (n // _R,)
    return pl.pallas_call(
        _body,
        grid=grid,
        in_specs=[pl.BlockSpec((_R, n), lambda i: (i, 0))],
        out_specs=pl.BlockSpec((_R, n), lambda i: (i, 0)),
        out_shape=jax.ShapeDtypeStruct((n, n), jnp.float32),
        compiler_params=pltpu.CompilerParams(
            dimension_semantics=("arbitrary",),
        ),
    )(adjacency)
